# baseline (speedup 1.0000x reference)
import functools

import jax
import jax.numpy as jnp
from jax import lax
from jax.experimental import pallas as pl
from jax.experimental.pallas import tpu as pltpu
from jax.experimental.pallas import tpu_sc as plsc

_SC_CORES = 2
_SC_SUBCORES = 16
_NW = _SC_CORES * _SC_SUBCORES
_CHUNK = 128


def _proj4_body(x_ref, w_ref, b_ref, o1_ref, o2_ref, o3_ref):
    x = x_ref[...]
    p = jnp.dot(x, w_ref[...], preferred_element_type=jnp.float32) + b_ref[...]
    d = x.shape[1]
    o1_ref[...] = p[:, :2 * d].astype(jnp.bfloat16)
    o2_ref[...] = p[:, 2 * d:3 * d].astype(jnp.bfloat16)
    o3_ref[...] = p[:, 3 * d:]


def _node_proj4(node_feats, W4, b4, block=1000):
    n, d = node_feats.shape
    k4 = W4.shape[1]
    return pl.pallas_call(
        _proj4_body,
        grid=(n // block,),
        in_specs=[
            pl.BlockSpec((block, d), lambda i: (i, 0)),
            pl.BlockSpec((d, k4), lambda i: (0, 0)),
            pl.BlockSpec((1, k4), lambda i: (0, 0)),
        ],
        out_specs=[
            pl.BlockSpec((block, 2 * d), lambda i: (i, 0)),
            pl.BlockSpec((block, d), lambda i: (i, 0)),
            pl.BlockSpec((block, d), lambda i: (i, 0)),
        ],
        out_shape=[
            jax.ShapeDtypeStruct((n, 2 * d), jnp.bfloat16),
            jax.ShapeDtypeStruct((n, d), jnp.bfloat16),
            jax.ShapeDtypeStruct((n, d), jnp.float32),
        ],
    )(node_feats, W4, b4)


def _layernorm(x, g, b, eps=1e-5):
    mu = jnp.mean(x, axis=-1, keepdims=True)
    var = jnp.mean((x - mu) ** 2, axis=-1, keepdims=True)
    return (x - mu) / jnp.sqrt(var + eps) * g + b


def _edge_body(ef_ref, gsb_ref, gd_ref, weg_ref, beg_ref, ge_ref,
               bee_ref, y_ref, sig_ref, t_ref):
    ef = ef_ref[...]
    d = ef.shape[1]
    gsb = gsb_ref[...].astype(jnp.float32)
    m = (gsb[:, :d] + gd_ref[...].astype(jnp.float32)
         + jnp.dot(ef, weg_ref[...], preferred_element_type=jnp.float32)
         + beg_ref[...])
    ln = _layernorm(m, ge_ref[...], bee_ref[...])
    y = ef + ln * jax.nn.sigmoid(ln)
    sig = jax.nn.sigmoid(y)
    y_ref[...] = y
    sig_ref[...] = sig
    t_ref[...] = gsb[:, d:] * sig


def _edge_stage(edge_feats, g_src_bh, g_dst, W_eg, b_eg, g_e, be_e,
                off=0, eh=None, block=2000):
    e, d = edge_feats.shape
    eh = e if eh is None else eh
    off_b = off // block
    row = lambda i: (i, 0)
    fixed = lambda i: (0, 0)
    out_sd = jax.ShapeDtypeStruct((eh, d), jnp.float32)
    return pl.pallas_call(
        _edge_body,
        grid=(eh // block,),
        in_specs=[
            pl.BlockSpec((block, d), lambda i: (i + off_b, 0)),
            pl.BlockSpec((block, 2 * d), row),
            pl.BlockSpec((block, d), row),
            pl.BlockSpec((d, d), fixed),
            pl.BlockSpec((1, d), fixed),
            pl.BlockSpec((1, d), fixed),
            pl.BlockSpec((1, d), fixed),
        ],
        out_specs=[
            pl.BlockSpec((block, d), row),
            pl.BlockSpec((block, d), row),
            pl.BlockSpec((block, d), row),
        ],
        out_shape=[out_sd, out_sd, out_sd],
    )(edge_feats, g_src_bh, g_dst, W_eg, b_eg, g_e, be_e)


def _node_out_body(nf_ref, xs_ref, ssh_ref, ss_ref, gn_ref, ben_ref, x_ref):
    h = ssh_ref[...] / (ss_ref[...] + 1e-6)
    v = xs_ref[...] + h
    ln = _layernorm(v, gn_ref[...], ben_ref[...])
    x_ref[...] = nf_ref[...] + ln * jax.nn.sigmoid(ln)


def _node_out_stage(node_feats, xs, ssh, ss, g_n, be_n, block=1000):
    n, d = node_feats.shape
    row = lambda i: (i, 0)
    fixed = lambda i: (0, 0)
    return pl.pallas_call(
        _node_out_body,
        grid=(n // block,),
        in_specs=[
            pl.BlockSpec((block, d), row),
            pl.BlockSpec((block, d), row),
            pl.BlockSpec((block, d), row),
            pl.BlockSpec((block, d), row),
            pl.BlockSpec((1, d), fixed),
            pl.BlockSpec((1, d), fixed),
        ],
        out_specs=pl.BlockSpec((block, d), row),
        out_shape=jax.ShapeDtypeStruct((n, d), jnp.float32),
    )(node_feats, xs, ssh, ss, g_n, be_n)


def _pad_rows(x, rows):
    if x.shape[0] >= rows:
        return x
    return jnp.concatenate(
        [x, jnp.zeros((rows - x.shape[0],) + x.shape[1:], x.dtype)], axis=0)


def _gather_split(idx, chunk):
    e = idx.shape[0]
    n_chunks = e // chunk
    per, rem = n_chunks // _NW, n_chunks % _NW
    maxc = per + (1 if rem else 0)
    if rem:
        idx = jnp.concatenate([idx, jnp.zeros((chunk,), idx.dtype)])
    return n_chunks, per, rem, maxc, idx


def _gather_phase(table_hbm, idx_hbm, out_hbm, idx_v, rows0, rows1,
                  gsem, osem0, osem1, chunk, per, rem, maxc):
    wid = lax.axis_index("s") * _SC_CORES + lax.axis_index("c")
    start = wid * per + jnp.minimum(wid, rem)
    count = per + jnp.where(wid < rem, 1, 0)
    pltpu.sync_copy(idx_hbm.at[pl.ds(start * chunk, maxc * chunk)], idx_v)
    npairs = (count + 1) // 2

    def out_slice(k_):
        return out_hbm.at[pl.ds((start + k_) * chunk, chunk)]

    @pl.loop(0, npairs)
    def _(p):
        k0 = 2 * p
        k1 = k0 + 1

        @pl.when(p > 0)
        def _():
            pltpu.make_async_copy(rows0, out_slice(0), osem0).wait()

        pltpu.async_copy(
            table_hbm.at[idx_v.at[pl.ds(k0 * chunk, chunk)]], rows0, gsem
        ).wait()
        pltpu.async_copy(rows0, out_slice(k0), osem0)

        @pl.when(k1 < count)
        def _():
            @pl.when(p > 0)
            def _():
                pltpu.make_async_copy(rows1, out_slice(0), osem1).wait()

            pltpu.async_copy(
                table_hbm.at[idx_v.at[pl.ds(k1 * chunk, chunk)]], rows1, gsem
            ).wait()
            pltpu.async_copy(rows1, out_slice(k1), osem1)

    @pl.when(count > 0)
    def _():
        pltpu.make_async_copy(rows0, out_slice(0), osem0).wait()

    @pl.when(count > 1)
    def _():
        pltpu.make_async_copy(rows1, out_slice(0), osem1).wait()


def _sc_gather(table, idx, chunk):
    e = idx.shape[0]
    n, d = table.shape
    _, per, rem, maxc, idx = _gather_split(idx, chunk)
    mesh = plsc.VectorSubcoreMesh(core_axis_name="c", subcore_axis_name="s")

    @functools.partial(
        pl.kernel,
        mesh=mesh,
        out_type=jax.ShapeDtypeStruct((e, d), table.dtype),
        scratch_types=[
            pltpu.VMEM((maxc * chunk,), jnp.int32),
            pltpu.VMEM((chunk, d), table.dtype),
            pltpu.VMEM((chunk, d), table.dtype),
            pltpu.SemaphoreType.DMA,
            pltpu.SemaphoreType.DMA,
            pltpu.SemaphoreType.DMA,
        ],
    )
    def k(t_hbm, i_hbm, o_hbm, i_v, r0, r1, gsem, osem0, osem1):
        _gather_phase(t_hbm, i_hbm, o_hbm, i_v, r0, r1,
                      gsem, osem0, osem1, chunk, per, rem, maxc)

    return k(table, idx)


def _sc_segment_sum(vals, idx, n):
    e, d = vals.shape
    dh = d // _SC_CORES
    vals3 = vals.reshape(e, _SC_CORES, dh)
    n_chunks = e // _CHUNK
    rows_per_sub = n // _SC_SUBCORES
    mesh = plsc.VectorSubcoreMesh(core_axis_name="c", subcore_axis_name="s")

    stride = max(-(-n_chunks // (_SC_SUBCORES * 8)) * 8, 8)
    idx2 = _pad_rows(idx.reshape(n_chunks, _CHUNK), _SC_SUBCORES * stride)
    zeros = jnp.zeros((rows_per_sub, dh), jnp.float32)

    @functools.partial(
        pl.kernel,
        mesh=mesh,
        out_type=jax.ShapeDtypeStruct((n, _SC_CORES, dh), jnp.float32),
        scratch_types=[
            pltpu.VMEM((stride, _CHUNK), jnp.int32),
            pltpu.VMEM((_CHUNK, dh), jnp.float32),
            pltpu.VMEM((_CHUNK, dh), jnp.float32),
            pltpu.VMEM_SHARED((n, dh), jnp.float32),
            pltpu.SemaphoreType.DMA,
            pltpu.SemaphoreType.DMA,
        ],
    )
    def k(vals_hbm, idx_hbm, z_hbm, out_hbm, idx_v, vals0, vals1, acc_sh,
          vsem0, vsem1):
        c = lax.axis_index("c")
        s = lax.axis_index("s")
        start = s * stride
        count = jnp.clip(n_chunks - start, 0, stride)
        my_rows = pl.ds(s * rows_per_sub, rows_per_sub)
        pltpu.sync_copy(idx_hbm.at[pl.ds(start, stride)], idx_v)
        pltpu.sync_copy(z_hbm, acc_sh.at[my_rows])
        plsc.subcore_barrier()

        def vals_slice(k_):
            return vals_hbm.at[pl.ds((start + k_) * _CHUNK, _CHUNK), c]

        @pl.when(count > 0)
        def _():
            pltpu.async_copy(vals_slice(0), vals0, vsem0)

        npairs = (count + 1) // 2

        @pl.loop(0, npairs)
        def _(p):
            k0 = 2 * p
            k1 = k0 + 1
            k2 = k0 + 2

            @pl.when(k1 < count)
            def _():
                pltpu.async_copy(vals_slice(k1), vals1, vsem1)

            pltpu.make_async_copy(vals_slice(0), vals0, vsem0).wait()
            pltpu.sync_copy(vals0, acc_sh.at[idx_v.at[k0]], add=True)

            @pl.when(k2 < count)
            def _():
                pltpu.async_copy(vals_slice(k2), vals0, vsem0)

            @pl.when(k1 < count)
            def _():
                pltpu.make_async_copy(vals_slice(0), vals1, vsem1).wait()
                pltpu.sync_copy(vals1, acc_sh.at[idx_v.at[k1]], add=True)

        plsc.subcore_barrier()
        pltpu.sync_copy(acc_sh.at[my_rows], out_hbm.at[my_rows, c])

    return k(vals3, idx2, zeros).reshape(n, d)


def kernel(node_feats, edge_feats, edge_index, W_sg, b_sg, W_dg, b_dg,
           W_eg, b_eg, W_su, b_su, W_du, b_du, g_e, be_e, g_n, be_n):
    n, d = node_feats.shape
    src = edge_index[0].astype(jnp.int32)
    dst = edge_index[1].astype(jnp.int32)

    W4 = jnp.concatenate([W_sg, W_du, W_dg, W_su], axis=1)
    b4 = jnp.concatenate([b_sg, b_du, b_dg, b_su]).reshape(1, 4 * d)
    src_tab, e_dst, xs = _node_proj4(node_feats, W4, b4)

    def to_i32(a):
        return lax.bitcast_convert_type(
            a.reshape(a.shape[0], a.shape[1] // 2, 2), jnp.int32)

    def to_bf16(a):
        return lax.bitcast_convert_type(a, jnp.bfloat16).reshape(
            a.shape[0], a.shape[1] * 2)

    g_src_bh = to_bf16(_sc_gather(to_i32(src_tab), src, 128))
    g_dst = to_bf16(_sc_gather(to_i32(e_dst), dst, 128))

    y, sigma, t = _edge_stage(
        edge_feats, g_src_bh, g_dst, W_eg,
        b_eg.reshape(1, d), g_e.reshape(1, d), be_e.reshape(1, d))

    ssh = _sc_segment_sum(t, dst, n)
    ss = _sc_segment_sum(sigma, dst, n)

    x = _node_out_stage(node_feats, xs, ssh, ss,
                        g_n.reshape(1, d), be_n.reshape(1, d))
    return (x, y)


# speedup vs baseline: 3.0493x; 3.0493x over previous
import functools

import jax
import jax.numpy as jnp
from jax import lax
from jax.experimental import pallas as pl
from jax.experimental.pallas import tpu as pltpu
from jax.experimental.pallas import tpu_sc as plsc

_SC_CORES = 2
_SC_SUBCORES = 16
_NW = _SC_CORES * _SC_SUBCORES
_CHUNK = 128


def _bf16_bits(x):
    b = lax.bitcast_convert_type(x, jnp.uint32)
    r = b + jnp.uint32(0x7FFF) + ((b >> 16) & jnp.uint32(1))
    return r & jnp.uint32(0xFFFF0000)


def _pack2(lo, hi):
    w = (_bf16_bits(lo) >> 16) | _bf16_bits(hi)
    return lax.bitcast_convert_type(w, jnp.int32)


def _unpack_lo(w):
    wu = lax.bitcast_convert_type(w, jnp.uint32)
    return lax.bitcast_convert_type(wu << 16, jnp.float32)


def _unpack_hi(w):
    wu = lax.bitcast_convert_type(w, jnp.uint32)
    return lax.bitcast_convert_type(wu & jnp.uint32(0xFFFF0000), jnp.float32)


def _proj4_body(x_ref, w_ref, b_ref, o1_ref, o2_ref, o3_ref):
    x = x_ref[...]
    p = jnp.dot(x, w_ref[...], preferred_element_type=jnp.float32) + b_ref[...]
    d = x.shape[1]
    dh = d // 2
    o1_ref[...] = _pack2(p[:, :d], p[:, d:2 * d])
    o2_ref[...] = _pack2(p[:, 2 * d:2 * d + dh], p[:, 2 * d + dh:3 * d])
    o3_ref[...] = p[:, 3 * d:]


def _node_proj4(node_feats, W4, b4, block=1000):
    n, d = node_feats.shape
    k4 = W4.shape[1]
    return pl.pallas_call(
        _proj4_body,
        grid=(n // block,),
        in_specs=[
            pl.BlockSpec((block, d), lambda i: (i, 0)),
            pl.BlockSpec((d, k4), lambda i: (0, 0)),
            pl.BlockSpec((1, k4), lambda i: (0, 0)),
        ],
        out_specs=[
            pl.BlockSpec((block, d), lambda i: (i, 0)),
            pl.BlockSpec((block, d // 2), lambda i: (i, 0)),
            pl.BlockSpec((block, d), lambda i: (i, 0)),
        ],
        out_shape=[
            jax.ShapeDtypeStruct((n, d), jnp.int32),
            jax.ShapeDtypeStruct((n, d // 2), jnp.int32),
            jax.ShapeDtypeStruct((n, d), jnp.float32),
        ],
    )(node_feats, W4, b4)


def _layernorm(x, g, b, eps=1e-5):
    mu = jnp.mean(x, axis=-1, keepdims=True)
    var = jnp.mean((x - mu) ** 2, axis=-1, keepdims=True)
    return (x - mu) / jnp.sqrt(var + eps) * g + b


def _edge_body(ef_ref, gsb_ref, gd_ref, weg_ref, beg_ref, ge_ref,
               bee_ref, y_ref, sig_ref, t_ref):
    ef = ef_ref[...]
    gsb = gsb_ref[...]
    gd = gd_ref[...]
    e_src = _unpack_lo(gsb)
    bh = _unpack_hi(gsb)
    e_dst = jnp.concatenate([_unpack_lo(gd), _unpack_hi(gd)], axis=1)
    m = (e_src + e_dst
         + jnp.dot(ef, weg_ref[...], preferred_element_type=jnp.float32)
         + beg_ref[...])
    ln = _layernorm(m, ge_ref[...], bee_ref[...])
    y = ef + ln * jax.nn.sigmoid(ln)
    sig = jax.nn.sigmoid(y)
    y_ref[...] = y
    sig_ref[...] = sig
    t_ref[...] = bh * sig


def _edge_stage(edge_feats, g_src_bh, g_dst, W_eg, b_eg, g_e, be_e,
                off=0, eh=None, block=2000):
    e, d = edge_feats.shape
    eh = e if eh is None else eh
    off_b = off // block
    row = lambda i: (i, 0)
    fixed = lambda i: (0, 0)
    out_sd = jax.ShapeDtypeStruct((eh, d), jnp.float32)
    return pl.pallas_call(
        _edge_body,
        grid=(eh // block,),
        in_specs=[
            pl.BlockSpec((block, d), lambda i: (i + off_b, 0)),
            pl.BlockSpec((block, d), row),
            pl.BlockSpec((block, d // 2), row),
            pl.BlockSpec((d, d), fixed),
            pl.BlockSpec((1, d), fixed),
            pl.BlockSpec((1, d), fixed),
            pl.BlockSpec((1, d), fixed),
        ],
        out_specs=[
            pl.BlockSpec((block, d), row),
            pl.BlockSpec((block, d), row),
            pl.BlockSpec((block, d), row),
        ],
        out_shape=[out_sd, out_sd, out_sd],
    )(edge_feats, g_src_bh, g_dst, W_eg, b_eg, g_e, be_e)


def _node_out_body(nf_ref, xs_ref, ssh_ref, ss_ref, gn_ref, ben_ref, x_ref):
    h = ssh_ref[...] / (ss_ref[...] + 1e-6)
    v = xs_ref[...] + h
    ln = _layernorm(v, gn_ref[...], ben_ref[...])
    x_ref[...] = nf_ref[...] + ln * jax.nn.sigmoid(ln)


def _node_out_stage(node_feats, xs, ssh, ss, g_n, be_n, block=1000):
    n, d = node_feats.shape
    row = lambda i: (i, 0)
    fixed = lambda i: (0, 0)
    return pl.pallas_call(
        _node_out_body,
        grid=(n // block,),
        in_specs=[
            pl.BlockSpec((block, d), row),
            pl.BlockSpec((block, d), row),
            pl.BlockSpec((block, d), row),
            pl.BlockSpec((block, d), row),
            pl.BlockSpec((1, d), fixed),
            pl.BlockSpec((1, d), fixed),
        ],
        out_specs=pl.BlockSpec((block, d), row),
        out_shape=jax.ShapeDtypeStruct((n, d), jnp.float32),
    )(node_feats, xs, ssh, ss, g_n, be_n)


def _pad_rows(x, rows):
    if x.shape[0] >= rows:
        return x
    return jnp.concatenate(
        [x, jnp.zeros((rows - x.shape[0],) + x.shape[1:], x.dtype)], axis=0)


def _gather_split(idx, chunk):
    e = idx.shape[0]
    n_chunks = e // chunk
    per, rem = n_chunks // _NW, n_chunks % _NW
    maxc = per + (1 if rem else 0)
    if rem:
        idx = jnp.concatenate([idx, jnp.zeros((chunk,), idx.dtype)])
    return n_chunks, per, rem, maxc, idx


def _gather_phase(table_hbm, idx_hbm, out_hbm, idx_v, rows0, rows1,
                  gsem, osem0, osem1, chunk, per, rem, maxc):
    wid = lax.axis_index("s") * _SC_CORES + lax.axis_index("c")
    start = wid * per + jnp.minimum(wid, rem)
    count = per + jnp.where(wid < rem, 1, 0)
    pltpu.sync_copy(idx_hbm.at[pl.ds(start * chunk, maxc * chunk)], idx_v)
    npairs = (count + 1) // 2

    def out_slice(k_):
        return out_hbm.at[pl.ds((start + k_) * chunk, chunk)]

    @pl.loop(0, npairs)
    def _(p):
        k0 = 2 * p
        k1 = k0 + 1

        @pl.when(p > 0)
        def _():
            pltpu.make_async_copy(rows0, out_slice(0), osem0).wait()

        pltpu.async_copy(
            table_hbm.at[idx_v.at[pl.ds(k0 * chunk, chunk)]], rows0, gsem
        ).wait()
        pltpu.async_copy(rows0, out_slice(k0), osem0)

        @pl.when(k1 < count)
        def _():
            @pl.when(p > 0)
            def _():
                pltpu.make_async_copy(rows1, out_slice(0), osem1).wait()

            pltpu.async_copy(
                table_hbm.at[idx_v.at[pl.ds(k1 * chunk, chunk)]], rows1, gsem
            ).wait()
            pltpu.async_copy(rows1, out_slice(k1), osem1)

    @pl.when(count > 0)
    def _():
        pltpu.make_async_copy(rows0, out_slice(0), osem0).wait()

    @pl.when(count > 1)
    def _():
        pltpu.make_async_copy(rows1, out_slice(0), osem1).wait()


def _sc_gather(table, idx, chunk):
    e = idx.shape[0]
    n, d = table.shape
    _, per, rem, maxc, idx = _gather_split(idx, chunk)
    mesh = plsc.VectorSubcoreMesh(core_axis_name="c", subcore_axis_name="s")

    @functools.partial(
        pl.kernel,
        mesh=mesh,
        out_type=jax.ShapeDtypeStruct((e, d), table.dtype),
        scratch_types=[
            pltpu.VMEM((maxc * chunk,), jnp.int32),
            pltpu.VMEM((chunk, d), table.dtype),
            pltpu.VMEM((chunk, d), table.dtype),
            pltpu.SemaphoreType.DMA,
            pltpu.SemaphoreType.DMA,
            pltpu.SemaphoreType.DMA,
        ],
    )
    def k(t_hbm, i_hbm, o_hbm, i_v, r0, r1, gsem, osem0, osem1):
        _gather_phase(t_hbm, i_hbm, o_hbm, i_v, r0, r1,
                      gsem, osem0, osem1, chunk, per, rem, maxc)

    return k(table, idx)


def _sc_segment_sum(vals, idx, n):
    e, d = vals.shape
    dh = d // _SC_CORES
    vals3 = vals.reshape(e, _SC_CORES, dh)
    n_chunks = e // _CHUNK
    rows_per_sub = n // _SC_SUBCORES
    mesh = plsc.VectorSubcoreMesh(core_axis_name="c", subcore_axis_name="s")

    stride = max(-(-n_chunks // (_SC_SUBCORES * 8)) * 8, 8)
    idx2 = _pad_rows(idx.reshape(n_chunks, _CHUNK), _SC_SUBCORES * stride)
    zeros = jnp.zeros((rows_per_sub, dh), jnp.float32)

    @functools.partial(
        pl.kernel,
        mesh=mesh,
        out_type=jax.ShapeDtypeStruct((n, _SC_CORES, dh), jnp.float32),
        scratch_types=[
            pltpu.VMEM((stride, _CHUNK), jnp.int32),
            pltpu.VMEM((_CHUNK, dh), jnp.float32),
            pltpu.VMEM((_CHUNK, dh), jnp.float32),
            pltpu.VMEM_SHARED((n, dh), jnp.float32),
            pltpu.SemaphoreType.DMA,
            pltpu.SemaphoreType.DMA,
        ],
    )
    def k(vals_hbm, idx_hbm, z_hbm, out_hbm, idx_v, vals0, vals1, acc_sh,
          vsem0, vsem1):
        c = lax.axis_index("c")
        s = lax.axis_index("s")
        start = s * stride
        count = jnp.clip(n_chunks - start, 0, stride)
        my_rows = pl.ds(s * rows_per_sub, rows_per_sub)
        pltpu.sync_copy(idx_hbm.at[pl.ds(start, stride)], idx_v)
        pltpu.sync_copy(z_hbm, acc_sh.at[my_rows])
        plsc.subcore_barrier()

        def vals_slice(k_):
            return vals_hbm.at[pl.ds((start + k_) * _CHUNK, _CHUNK), c]

        @pl.when(count > 0)
        def _():
            pltpu.async_copy(vals_slice(0), vals0, vsem0)

        npairs = (count + 1) // 2

        @pl.loop(0, npairs)
        def _(p):
            k0 = 2 * p
            k1 = k0 + 1
            k2 = k0 + 2

            @pl.when(k1 < count)
            def _():
                pltpu.async_copy(vals_slice(k1), vals1, vsem1)

            pltpu.make_async_copy(vals_slice(0), vals0, vsem0).wait()
            pltpu.sync_copy(vals0, acc_sh.at[idx_v.at[k0]], add=True)

            @pl.when(k2 < count)
            def _():
                pltpu.async_copy(vals_slice(k2), vals0, vsem0)

            @pl.when(k1 < count)
            def _():
                pltpu.make_async_copy(vals_slice(0), vals1, vsem1).wait()
                pltpu.sync_copy(vals1, acc_sh.at[idx_v.at[k1]], add=True)

        plsc.subcore_barrier()
        pltpu.sync_copy(acc_sh.at[my_rows], out_hbm.at[my_rows, c])

    return k(vals3, idx2, zeros).reshape(n, d)


def kernel(node_feats, edge_feats, edge_index, W_sg, b_sg, W_dg, b_dg,
           W_eg, b_eg, W_su, b_su, W_du, b_du, g_e, be_e, g_n, be_n):
    n, d = node_feats.shape
    src = edge_index[0].astype(jnp.int32)
    dst = edge_index[1].astype(jnp.int32)

    W4 = jnp.concatenate([W_sg, W_du, W_dg, W_su], axis=1)
    b4 = jnp.concatenate([b_sg, b_du, b_dg, b_su]).reshape(1, 4 * d)
    src_tab, e_dst, xs = _node_proj4(node_feats, W4, b4)

    g_src_bh = _sc_gather(src_tab, src, 128)
    g_dst = _sc_gather(e_dst, dst, 128)

    y, sigma, t = _edge_stage(
        edge_feats, g_src_bh, g_dst, W_eg,
        b_eg.reshape(1, d), g_e.reshape(1, d), be_e.reshape(1, d))

    ssh = _sc_segment_sum(t, dst, n)
    ss = _sc_segment_sum(sigma, dst, n)

    x = _node_out_stage(node_feats, xs, ssh, ss,
                        g_n.reshape(1, d), be_n.reshape(1, d))
    return (x, y)


# speedup vs baseline: 3.0652x; 1.0052x over previous
import functools

import jax
import jax.numpy as jnp
from jax import lax
from jax.experimental import pallas as pl
from jax.experimental.pallas import tpu as pltpu
from jax.experimental.pallas import tpu_sc as plsc

_SC_CORES = 2
_SC_SUBCORES = 16
_NW = _SC_CORES * _SC_SUBCORES
_CHUNK = 128


def _bf16_bits(x):
    b = lax.bitcast_convert_type(x, jnp.uint32)
    r = b + jnp.uint32(0x7FFF) + ((b >> 16) & jnp.uint32(1))
    return r & jnp.uint32(0xFFFF0000)


def _pack2(lo, hi):
    w = (_bf16_bits(lo) >> 16) | _bf16_bits(hi)
    return lax.bitcast_convert_type(w, jnp.int32)


def _unpack_lo(w):
    wu = lax.bitcast_convert_type(w, jnp.uint32)
    return lax.bitcast_convert_type(wu << 16, jnp.float32)


def _unpack_hi(w):
    wu = lax.bitcast_convert_type(w, jnp.uint32)
    return lax.bitcast_convert_type(wu & jnp.uint32(0xFFFF0000), jnp.float32)


def _proj4_body(x_ref, w_ref, b_ref, o1_ref, o2_ref, o3_ref):
    x = x_ref[...]
    p = jnp.dot(x, w_ref[...], preferred_element_type=jnp.float32) + b_ref[...]
    d = x.shape[1]
    dh = d // 2
    o1_ref[...] = _pack2(p[:, :d], p[:, d:2 * d])
    o2_ref[...] = _pack2(p[:, 2 * d:2 * d + dh], p[:, 2 * d + dh:3 * d])
    o3_ref[...] = p[:, 3 * d:]


def _node_proj4(node_feats, W4, b4, block=1000):
    n, d = node_feats.shape
    k4 = W4.shape[1]
    return pl.pallas_call(
        _proj4_body,
        grid=(n // block,),
        in_specs=[
            pl.BlockSpec((block, d), lambda i: (i, 0)),
            pl.BlockSpec((d, k4), lambda i: (0, 0)),
            pl.BlockSpec((1, k4), lambda i: (0, 0)),
        ],
        out_specs=[
            pl.BlockSpec((block, d), lambda i: (i, 0)),
            pl.BlockSpec((block, d // 2), lambda i: (i, 0)),
            pl.BlockSpec((block, d), lambda i: (i, 0)),
        ],
        out_shape=[
            jax.ShapeDtypeStruct((n, d), jnp.int32),
            jax.ShapeDtypeStruct((n, d // 2), jnp.int32),
            jax.ShapeDtypeStruct((n, d), jnp.float32),
        ],
    )(node_feats, W4, b4)


def _layernorm(x, g, b, eps=1e-5):
    mu = jnp.mean(x, axis=-1, keepdims=True)
    var = jnp.mean((x - mu) ** 2, axis=-1, keepdims=True)
    return (x - mu) / jnp.sqrt(var + eps) * g + b


def _edge_body(ef_ref, gsb_ref, gd_ref, weg_ref, beg_ref, ge_ref,
               bee_ref, y_ref, sig_ref, t_ref):
    ef = ef_ref[...]
    gsb = gsb_ref[...]
    gd = gd_ref[...]
    e_src = _unpack_lo(gsb)
    bh = _unpack_hi(gsb)
    e_dst = jnp.concatenate([_unpack_lo(gd), _unpack_hi(gd)], axis=1)
    m = (e_src + e_dst
         + jnp.dot(ef.astype(jnp.bfloat16), weg_ref[...].astype(jnp.bfloat16),
                   preferred_element_type=jnp.float32)
         + beg_ref[...])
    ln = _layernorm(m, ge_ref[...], bee_ref[...])
    y = ef + ln * jax.nn.sigmoid(ln)
    sig = jax.nn.sigmoid(y)
    y_ref[...] = y
    sig_ref[...] = sig
    t_ref[...] = bh * sig


def _edge_stage(edge_feats, g_src_bh, g_dst, W_eg, b_eg, g_e, be_e,
                off=0, eh=None, block=4000):
    e, d = edge_feats.shape
    eh = e if eh is None else eh
    off_b = off // block
    row = lambda i: (i, 0)
    fixed = lambda i: (0, 0)
    out_sd = jax.ShapeDtypeStruct((eh, d), jnp.float32)
    return pl.pallas_call(
        _edge_body,
        grid=(eh // block,),
        in_specs=[
            pl.BlockSpec((block, d), lambda i: (i + off_b, 0)),
            pl.BlockSpec((block, d), row),
            pl.BlockSpec((block, d // 2), row),
            pl.BlockSpec((d, d), fixed),
            pl.BlockSpec((1, d), fixed),
            pl.BlockSpec((1, d), fixed),
            pl.BlockSpec((1, d), fixed),
        ],
        out_specs=[
            pl.BlockSpec((block, d), row),
            pl.BlockSpec((block, d), row),
            pl.BlockSpec((block, d), row),
        ],
        out_shape=[out_sd, out_sd, out_sd],
    )(edge_feats, g_src_bh, g_dst, W_eg, b_eg, g_e, be_e)


def _node_out_body(nf_ref, xs_ref, ssh_ref, ss_ref, gn_ref, ben_ref, x_ref):
    h = ssh_ref[...] / (ss_ref[...] + 1e-6)
    v = xs_ref[...] + h
    ln = _layernorm(v, gn_ref[...], ben_ref[...])
    x_ref[...] = nf_ref[...] + ln * jax.nn.sigmoid(ln)


def _node_out_stage(node_feats, xs, ssh, ss, g_n, be_n, block=1000):
    n, d = node_feats.shape
    row = lambda i: (i, 0)
    fixed = lambda i: (0, 0)
    return pl.pallas_call(
        _node_out_body,
        grid=(n // block,),
        in_specs=[
            pl.BlockSpec((block, d), row),
            pl.BlockSpec((block, d), row),
            pl.BlockSpec((block, d), row),
            pl.BlockSpec((block, d), row),
            pl.BlockSpec((1, d), fixed),
            pl.BlockSpec((1, d), fixed),
        ],
        out_specs=pl.BlockSpec((block, d), row),
        out_shape=jax.ShapeDtypeStruct((n, d), jnp.float32),
    )(node_feats, xs, ssh, ss, g_n, be_n)


def _pad_rows(x, rows):
    if x.shape[0] >= rows:
        return x
    return jnp.concatenate(
        [x, jnp.zeros((rows - x.shape[0],) + x.shape[1:], x.dtype)], axis=0)


def _gather_split(idx, chunk):
    e = idx.shape[0]
    n_chunks = e // chunk
    per, rem = n_chunks // _NW, n_chunks % _NW
    maxc = per + (1 if rem else 0)
    if rem:
        idx = jnp.concatenate([idx, jnp.zeros((chunk,), idx.dtype)])
    return n_chunks, per, rem, maxc, idx


def _gather_phase(table_hbm, idx_hbm, out_hbm, idx_v, rows0, rows1,
                  gsem, osem0, osem1, chunk, per, rem, maxc):
    wid = lax.axis_index("s") * _SC_CORES + lax.axis_index("c")
    start = wid * per + jnp.minimum(wid, rem)
    count = per + jnp.where(wid < rem, 1, 0)
    pltpu.sync_copy(idx_hbm.at[pl.ds(start * chunk, maxc * chunk)], idx_v)
    npairs = (count + 1) // 2

    def out_slice(k_):
        return out_hbm.at[pl.ds((start + k_) * chunk, chunk)]

    @pl.loop(0, npairs)
    def _(p):
        k0 = 2 * p
        k1 = k0 + 1

        @pl.when(p > 0)
        def _():
            pltpu.make_async_copy(rows0, out_slice(0), osem0).wait()

        pltpu.async_copy(
            table_hbm.at[idx_v.at[pl.ds(k0 * chunk, chunk)]], rows0, gsem
        ).wait()
        pltpu.async_copy(rows0, out_slice(k0), osem0)

        @pl.when(k1 < count)
        def _():
            @pl.when(p > 0)
            def _():
                pltpu.make_async_copy(rows1, out_slice(0), osem1).wait()

            pltpu.async_copy(
                table_hbm.at[idx_v.at[pl.ds(k1 * chunk, chunk)]], rows1, gsem
            ).wait()
            pltpu.async_copy(rows1, out_slice(k1), osem1)

    @pl.when(count > 0)
    def _():
        pltpu.make_async_copy(rows0, out_slice(0), osem0).wait()

    @pl.when(count > 1)
    def _():
        pltpu.make_async_copy(rows1, out_slice(0), osem1).wait()


def _sc_gather(table, idx, chunk):
    e = idx.shape[0]
    n, d = table.shape
    _, per, rem, maxc, idx = _gather_split(idx, chunk)
    mesh = plsc.VectorSubcoreMesh(core_axis_name="c", subcore_axis_name="s")

    @functools.partial(
        pl.kernel,
        mesh=mesh,
        out_type=jax.ShapeDtypeStruct((e, d), table.dtype),
        scratch_types=[
            pltpu.VMEM((maxc * chunk,), jnp.int32),
            pltpu.VMEM((chunk, d), table.dtype),
            pltpu.VMEM((chunk, d), table.dtype),
            pltpu.SemaphoreType.DMA,
            pltpu.SemaphoreType.DMA,
            pltpu.SemaphoreType.DMA,
        ],
    )
    def k(t_hbm, i_hbm, o_hbm, i_v, r0, r1, gsem, osem0, osem1):
        _gather_phase(t_hbm, i_hbm, o_hbm, i_v, r0, r1,
                      gsem, osem0, osem1, chunk, per, rem, maxc)

    return k(table, idx)


def _sc_segment_sum(vals, idx, n):
    e, d = vals.shape
    dh = d // _SC_CORES
    vals3 = vals.reshape(e, _SC_CORES, dh)
    n_chunks = e // _CHUNK
    rows_per_sub = n // _SC_SUBCORES
    mesh = plsc.VectorSubcoreMesh(core_axis_name="c", subcore_axis_name="s")

    stride = max(-(-n_chunks // (_SC_SUBCORES * 8)) * 8, 8)
    idx2 = _pad_rows(idx.reshape(n_chunks, _CHUNK), _SC_SUBCORES * stride)
    zeros = jnp.zeros((rows_per_sub, dh), jnp.float32)

    @functools.partial(
        pl.kernel,
        mesh=mesh,
        out_type=jax.ShapeDtypeStruct((n, _SC_CORES, dh), jnp.float32),
        scratch_types=[
            pltpu.VMEM((stride, _CHUNK), jnp.int32),
            pltpu.VMEM((_CHUNK, dh), jnp.float32),
            pltpu.VMEM((_CHUNK, dh), jnp.float32),
            pltpu.VMEM_SHARED((n, dh), jnp.float32),
            pltpu.SemaphoreType.DMA,
            pltpu.SemaphoreType.DMA,
        ],
    )
    def k(vals_hbm, idx_hbm, z_hbm, out_hbm, idx_v, vals0, vals1, acc_sh,
          vsem0, vsem1):
        c = lax.axis_index("c")
        s = lax.axis_index("s")
        start = s * stride
        count = jnp.clip(n_chunks - start, 0, stride)
        my_rows = pl.ds(s * rows_per_sub, rows_per_sub)
        pltpu.sync_copy(idx_hbm.at[pl.ds(start, stride)], idx_v)
        pltpu.sync_copy(z_hbm, acc_sh.at[my_rows])
        plsc.subcore_barrier()

        def vals_slice(k_):
            return vals_hbm.at[pl.ds((start + k_) * _CHUNK, _CHUNK), c]

        @pl.when(count > 0)
        def _():
            pltpu.async_copy(vals_slice(0), vals0, vsem0)

        npairs = (count + 1) // 2

        @pl.loop(0, npairs)
        def _(p):
            k0 = 2 * p
            k1 = k0 + 1
            k2 = k0 + 2

            @pl.when(k1 < count)
            def _():
                pltpu.async_copy(vals_slice(k1), vals1, vsem1)

            pltpu.make_async_copy(vals_slice(0), vals0, vsem0).wait()
            pltpu.sync_copy(vals0, acc_sh.at[idx_v.at[k0]], add=True)

            @pl.when(k2 < count)
            def _():
                pltpu.async_copy(vals_slice(k2), vals0, vsem0)

            @pl.when(k1 < count)
            def _():
                pltpu.make_async_copy(vals_slice(0), vals1, vsem1).wait()
                pltpu.sync_copy(vals1, acc_sh.at[idx_v.at[k1]], add=True)

        plsc.subcore_barrier()
        pltpu.sync_copy(acc_sh.at[my_rows], out_hbm.at[my_rows, c])

    return k(vals3, idx2, zeros).reshape(n, d)


def kernel(node_feats, edge_feats, edge_index, W_sg, b_sg, W_dg, b_dg,
           W_eg, b_eg, W_su, b_su, W_du, b_du, g_e, be_e, g_n, be_n):
    n, d = node_feats.shape
    src = edge_index[0].astype(jnp.int32)
    dst = edge_index[1].astype(jnp.int32)

    W4 = jnp.concatenate([W_sg, W_du, W_dg, W_su], axis=1)
    b4 = jnp.concatenate([b_sg, b_du, b_dg, b_su]).reshape(1, 4 * d)
    src_tab, e_dst, xs = _node_proj4(node_feats, W4, b4)

    g_src_bh = _sc_gather(src_tab, src, 128)
    g_dst = _sc_gather(e_dst, dst, 128)

    y, sigma, t = _edge_stage(
        edge_feats, g_src_bh, g_dst, W_eg,
        b_eg.reshape(1, d), g_e.reshape(1, d), be_e.reshape(1, d))

    ssh = _sc_segment_sum(t, dst, n)
    ss = _sc_segment_sum(sigma, dst, n)

    x = _node_out_stage(node_feats, xs, ssh, ss,
                        g_n.reshape(1, d), be_n.reshape(1, d))
    return (x, y)


# speedup vs baseline: 3.0673x; 1.0007x over previous
import functools

import jax
import jax.numpy as jnp
from jax import lax
from jax.experimental import pallas as pl
from jax.experimental.pallas import tpu as pltpu
from jax.experimental.pallas import tpu_sc as plsc

_SC_CORES = 2
_SC_SUBCORES = 16
_NW = _SC_CORES * _SC_SUBCORES
_CHUNK = 128


def _bf16_bits(x):
    b = lax.bitcast_convert_type(x, jnp.uint32)
    r = b + jnp.uint32(0x7FFF) + ((b >> 16) & jnp.uint32(1))
    return r & jnp.uint32(0xFFFF0000)


def _pack2(lo, hi):
    w = (_bf16_bits(lo) >> 16) | _bf16_bits(hi)
    return lax.bitcast_convert_type(w, jnp.int32)


def _unpack_lo(w):
    wu = lax.bitcast_convert_type(w, jnp.uint32)
    return lax.bitcast_convert_type(wu << 16, jnp.float32)


def _unpack_hi(w):
    wu = lax.bitcast_convert_type(w, jnp.uint32)
    return lax.bitcast_convert_type(wu & jnp.uint32(0xFFFF0000), jnp.float32)


def _proj4_body(x_ref, w_ref, b_ref, o1_ref, o2_ref, o3_ref):
    x = x_ref[...]
    p = jnp.dot(x, w_ref[...], preferred_element_type=jnp.float32) + b_ref[...]
    d = x.shape[1]
    dh = d // 2
    o1_ref[...] = _pack2(p[:, :d], p[:, d:2 * d])
    o2_ref[...] = _pack2(p[:, 2 * d:2 * d + dh], p[:, 2 * d + dh:3 * d])
    o3_ref[...] = p[:, 3 * d:]


def _node_proj4(node_feats, W4, b4, block=1000):
    n, d = node_feats.shape
    k4 = W4.shape[1]
    return pl.pallas_call(
        _proj4_body,
        grid=(n // block,),
        in_specs=[
            pl.BlockSpec((block, d), lambda i: (i, 0)),
            pl.BlockSpec((d, k4), lambda i: (0, 0)),
            pl.BlockSpec((1, k4), lambda i: (0, 0)),
        ],
        out_specs=[
            pl.BlockSpec((block, d), lambda i: (i, 0)),
            pl.BlockSpec((block, d // 2), lambda i: (i, 0)),
            pl.BlockSpec((block, d), lambda i: (i, 0)),
        ],
        out_shape=[
            jax.ShapeDtypeStruct((n, d), jnp.int32),
            jax.ShapeDtypeStruct((n, d // 2), jnp.int32),
            jax.ShapeDtypeStruct((n, d), jnp.float32),
        ],
    )(node_feats, W4, b4)


def _layernorm(x, g, b, eps=1e-5):
    mu = jnp.mean(x, axis=-1, keepdims=True)
    var = jnp.mean((x - mu) ** 2, axis=-1, keepdims=True)
    return (x - mu) / jnp.sqrt(var + eps) * g + b


def _edge_body(ef_ref, gsb_ref, gd_ref, weg_ref, beg_ref, ge_ref,
               bee_ref, y_ref, sig_ref, t_ref):
    ef = ef_ref[...]
    gsb = gsb_ref[...]
    gd = gd_ref[...]
    e_src = _unpack_lo(gsb)
    bh = _unpack_hi(gsb)
    e_dst = jnp.concatenate([_unpack_lo(gd), _unpack_hi(gd)], axis=1)
    m = (e_src + e_dst
         + jnp.dot(ef.astype(jnp.bfloat16), weg_ref[...].astype(jnp.bfloat16),
                   preferred_element_type=jnp.float32)
         + beg_ref[...])
    ln = _layernorm(m, ge_ref[...], bee_ref[...])
    y = ef + ln * jax.nn.sigmoid(ln)
    sig = jax.nn.sigmoid(y)
    y_ref[...] = y
    sig_ref[...] = sig
    t_ref[...] = bh * sig


def _edge_stage(edge_feats, g_src_bh, g_dst, W_eg, b_eg, g_e, be_e,
                block=4000):
    e, d = edge_feats.shape
    row = lambda i: (i, 0)
    fixed = lambda i: (0, 0)
    out_sd = jax.ShapeDtypeStruct((e, d), jnp.float32)
    return pl.pallas_call(
        _edge_body,
        grid=(e // block,),
        in_specs=[
            pl.BlockSpec((block, d), row),
            pl.BlockSpec((block, d), row),
            pl.BlockSpec((block, d // 2), row),
            pl.BlockSpec((d, d), fixed),
            pl.BlockSpec((1, d), fixed),
            pl.BlockSpec((1, d), fixed),
            pl.BlockSpec((1, d), fixed),
        ],
        out_specs=[
            pl.BlockSpec((block, d), row),
            pl.BlockSpec((block, d), row),
            pl.BlockSpec((block, d), row),
        ],
        out_shape=[out_sd, out_sd, out_sd],
    )(edge_feats, g_src_bh, g_dst, W_eg, b_eg, g_e, be_e)


def _node_out_body(nf_ref, xs_ref, ssh_ref, ss_ref, gn_ref, ben_ref, x_ref):
    h = ssh_ref[...] / (ss_ref[...] + 1e-6)
    v = xs_ref[...] + h
    ln = _layernorm(v, gn_ref[...], ben_ref[...])
    x_ref[...] = nf_ref[...] + ln * jax.nn.sigmoid(ln)


def _node_out_stage(node_feats, xs, ssh, ss, g_n, be_n, block=1000):
    n, d = node_feats.shape
    row = lambda i: (i, 0)
    fixed = lambda i: (0, 0)
    return pl.pallas_call(
        _node_out_body,
        grid=(n // block,),
        in_specs=[
            pl.BlockSpec((block, d), row),
            pl.BlockSpec((block, d), row),
            pl.BlockSpec((block, d), row),
            pl.BlockSpec((block, d), row),
            pl.BlockSpec((1, d), fixed),
            pl.BlockSpec((1, d), fixed),
        ],
        out_specs=pl.BlockSpec((block, d), row),
        out_shape=jax.ShapeDtypeStruct((n, d), jnp.float32),
    )(node_feats, xs, ssh, ss, g_n, be_n)


def _pad_rows(x, rows):
    if x.shape[0] >= rows:
        return x
    return jnp.concatenate(
        [x, jnp.zeros((rows - x.shape[0],) + x.shape[1:], x.dtype)], axis=0)


def _gather_split(idx, chunk):
    e = idx.shape[0]
    n_chunks = e // chunk
    per, rem = n_chunks // _NW, n_chunks % _NW
    maxc = per + (1 if rem else 0)
    if rem:
        idx = jnp.concatenate([idx, jnp.zeros((chunk,), idx.dtype)])
    return n_chunks, per, rem, maxc, idx


def _gather_phase(table_hbm, idx_hbm, out_hbm, idx_v, rows0, rows1,
                  gsem, osem0, osem1, chunk, per, rem, maxc):
    wid = lax.axis_index("s") * _SC_CORES + lax.axis_index("c")
    start = wid * per + jnp.minimum(wid, rem)
    count = per + jnp.where(wid < rem, 1, 0)
    pltpu.sync_copy(idx_hbm.at[pl.ds(start * chunk, maxc * chunk)], idx_v)
    npairs = (count + 1) // 2

    def out_slice(k_):
        return out_hbm.at[pl.ds((start + k_) * chunk, chunk)]

    @pl.loop(0, npairs)
    def _(p):
        k0 = 2 * p
        k1 = k0 + 1

        @pl.when(p > 0)
        def _():
            pltpu.make_async_copy(rows0, out_slice(0), osem0).wait()

        pltpu.async_copy(
            table_hbm.at[idx_v.at[pl.ds(k0 * chunk, chunk)]], rows0, gsem
        ).wait()
        pltpu.async_copy(rows0, out_slice(k0), osem0)

        @pl.when(k1 < count)
        def _():
            @pl.when(p > 0)
            def _():
                pltpu.make_async_copy(rows1, out_slice(0), osem1).wait()

            pltpu.async_copy(
                table_hbm.at[idx_v.at[pl.ds(k1 * chunk, chunk)]], rows1, gsem
            ).wait()
            pltpu.async_copy(rows1, out_slice(k1), osem1)

    @pl.when(count > 0)
    def _():
        pltpu.make_async_copy(rows0, out_slice(0), osem0).wait()

    @pl.when(count > 1)
    def _():
        pltpu.make_async_copy(rows1, out_slice(0), osem1).wait()


def _sc_gather(table, idx, chunk):
    e = idx.shape[0]
    n, d = table.shape
    _, per, rem, maxc, idx = _gather_split(idx, chunk)
    mesh = plsc.VectorSubcoreMesh(core_axis_name="c", subcore_axis_name="s")

    @functools.partial(
        pl.kernel,
        mesh=mesh,
        out_type=jax.ShapeDtypeStruct((e, d), table.dtype),
        scratch_types=[
            pltpu.VMEM((maxc * chunk,), jnp.int32),
            pltpu.VMEM((chunk, d), table.dtype),
            pltpu.VMEM((chunk, d), table.dtype),
            pltpu.SemaphoreType.DMA,
            pltpu.SemaphoreType.DMA,
            pltpu.SemaphoreType.DMA,
        ],
    )
    def k(t_hbm, i_hbm, o_hbm, i_v, r0, r1, gsem, osem0, osem1):
        _gather_phase(t_hbm, i_hbm, o_hbm, i_v, r0, r1,
                      gsem, osem0, osem1, chunk, per, rem, maxc)

    return k(table, idx)


def _sc_segment_sum(vals, idx, n):
    e, d = vals.shape
    dh = d // _SC_CORES
    vals3 = vals.reshape(e, _SC_CORES, dh)
    n_chunks = e // _CHUNK
    rows_per_sub = n // _SC_SUBCORES
    mesh = plsc.VectorSubcoreMesh(core_axis_name="c", subcore_axis_name="s")

    stride = max(-(-n_chunks // (_SC_SUBCORES * 8)) * 8, 8)
    idx2 = _pad_rows(idx.reshape(n_chunks, _CHUNK), _SC_SUBCORES * stride)
    zeros = jnp.zeros((rows_per_sub, dh), jnp.float32)

    @functools.partial(
        pl.kernel,
        mesh=mesh,
        out_type=jax.ShapeDtypeStruct((n, _SC_CORES, dh), jnp.float32),
        scratch_types=[
            pltpu.VMEM((stride, _CHUNK), jnp.int32),
            pltpu.VMEM((_CHUNK, dh), jnp.float32),
            pltpu.VMEM((_CHUNK, dh), jnp.float32),
            pltpu.VMEM_SHARED((n, dh), jnp.float32),
            pltpu.SemaphoreType.DMA,
            pltpu.SemaphoreType.DMA,
        ],
    )
    def k(vals_hbm, idx_hbm, z_hbm, out_hbm, idx_v, vals0, vals1, acc_sh,
          vsem0, vsem1):
        c = lax.axis_index("c")
        s = lax.axis_index("s")
        start = s * stride
        count = jnp.clip(n_chunks - start, 0, stride)
        my_rows = pl.ds(s * rows_per_sub, rows_per_sub)
        pltpu.sync_copy(idx_hbm.at[pl.ds(start, stride)], idx_v)
        pltpu.sync_copy(z_hbm, acc_sh.at[my_rows])
        plsc.subcore_barrier()

        def vals_slice(k_):
            return vals_hbm.at[pl.ds((start + k_) * _CHUNK, _CHUNK), c]

        @pl.when(count > 0)
        def _():
            pltpu.async_copy(vals_slice(0), vals0, vsem0)

        npairs = (count + 1) // 2

        @pl.loop(0, npairs)
        def _(p):
            k0 = 2 * p
            k1 = k0 + 1
            k2 = k0 + 2

            @pl.when(k1 < count)
            def _():
                pltpu.async_copy(vals_slice(k1), vals1, vsem1)

            pltpu.make_async_copy(vals_slice(0), vals0, vsem0).wait()
            pltpu.sync_copy(vals0, acc_sh.at[idx_v.at[k0]], add=True)

            @pl.when(k2 < count)
            def _():
                pltpu.async_copy(vals_slice(k2), vals0, vsem0)

            @pl.when(k1 < count)
            def _():
                pltpu.make_async_copy(vals_slice(0), vals1, vsem1).wait()
                pltpu.sync_copy(vals1, acc_sh.at[idx_v.at[k1]], add=True)

        plsc.subcore_barrier()
        pltpu.sync_copy(acc_sh.at[my_rows], out_hbm.at[my_rows, c])

    return k(vals3, idx2, zeros).reshape(n, d)


def kernel(node_feats, edge_feats, edge_index, W_sg, b_sg, W_dg, b_dg,
           W_eg, b_eg, W_su, b_su, W_du, b_du, g_e, be_e, g_n, be_n):
    n, d = node_feats.shape
    src = edge_index[0].astype(jnp.int32)
    dst = edge_index[1].astype(jnp.int32)

    W4 = jnp.concatenate([W_sg, W_du, W_dg, W_su], axis=1)
    b4 = jnp.concatenate([b_sg, b_du, b_dg, b_su]).reshape(1, 4 * d)
    src_tab, e_dst, xs = _node_proj4(node_feats, W4, b4)

    g_src_bh = _sc_gather(src_tab, src, 128)
    g_dst = _sc_gather(e_dst, dst, 128)

    y, sigma, t = _edge_stage(
        edge_feats, g_src_bh, g_dst, W_eg,
        b_eg.reshape(1, d), g_e.reshape(1, d), be_e.reshape(1, d))

    ssh = _sc_segment_sum(t, dst, n)
    ss = _sc_segment_sum(sigma, dst, n)

    x = _node_out_stage(node_feats, xs, ssh, ss,
                        g_n.reshape(1, d), be_n.reshape(1, d))
    return (x, y)


# speedup vs baseline: 3.0760x; 1.0028x over previous
import functools

import jax
import jax.numpy as jnp
from jax import lax
from jax.experimental import pallas as pl
from jax.experimental.pallas import tpu as pltpu
from jax.experimental.pallas import tpu_sc as plsc

_SC_CORES = 2
_SC_SUBCORES = 16
_NW = _SC_CORES * _SC_SUBCORES
_CHUNK = 128


def _bf16_bits(x):
    b = lax.bitcast_convert_type(x, jnp.uint32)
    r = b + jnp.uint32(0x7FFF) + ((b >> 16) & jnp.uint32(1))
    return r & jnp.uint32(0xFFFF0000)


def _pack2(lo, hi):
    w = (_bf16_bits(lo) >> 16) | _bf16_bits(hi)
    return lax.bitcast_convert_type(w, jnp.int32)


def _unpack_lo(w):
    wu = lax.bitcast_convert_type(w, jnp.uint32)
    return lax.bitcast_convert_type(wu << 16, jnp.float32)


def _unpack_hi(w):
    wu = lax.bitcast_convert_type(w, jnp.uint32)
    return lax.bitcast_convert_type(wu & jnp.uint32(0xFFFF0000), jnp.float32)


def _proj4_body(x_ref, w_ref, b_ref, o1_ref, o2_ref, o3_ref):
    x = x_ref[...]
    p = jnp.dot(x, w_ref[...], preferred_element_type=jnp.float32) + b_ref[...]
    d = x.shape[1]
    dh = d // 2
    o1_ref[...] = _pack2(p[:, :d], p[:, d:2 * d])
    o2_ref[...] = _pack2(p[:, 2 * d:2 * d + dh], p[:, 2 * d + dh:3 * d])
    o3_ref[...] = p[:, 3 * d:]


def _node_proj4(node_feats, W4, b4, block=2000):
    n, d = node_feats.shape
    k4 = W4.shape[1]
    return pl.pallas_call(
        _proj4_body,
        grid=(n // block,),
        in_specs=[
            pl.BlockSpec((block, d), lambda i: (i, 0)),
            pl.BlockSpec((d, k4), lambda i: (0, 0)),
            pl.BlockSpec((1, k4), lambda i: (0, 0)),
        ],
        out_specs=[
            pl.BlockSpec((block, d), lambda i: (i, 0)),
            pl.BlockSpec((block, d // 2), lambda i: (i, 0)),
            pl.BlockSpec((block, d), lambda i: (i, 0)),
        ],
        out_shape=[
            jax.ShapeDtypeStruct((n, d), jnp.int32),
            jax.ShapeDtypeStruct((n, d // 2), jnp.int32),
            jax.ShapeDtypeStruct((n, d), jnp.float32),
        ],
    )(node_feats, W4, b4)


def _layernorm(x, g, b, eps=1e-5):
    mu = jnp.mean(x, axis=-1, keepdims=True)
    var = jnp.mean((x - mu) ** 2, axis=-1, keepdims=True)
    return (x - mu) / jnp.sqrt(var + eps) * g + b


def _edge_body(ef_ref, gsb_ref, gd_ref, weg_ref, beg_ref, ge_ref,
               bee_ref, y_ref, sig_ref, t_ref):
    ef = ef_ref[...]
    gsb = gsb_ref[...]
    gd = gd_ref[...]
    e_src = _unpack_lo(gsb)
    bh = _unpack_hi(gsb)
    e_dst = jnp.concatenate([_unpack_lo(gd), _unpack_hi(gd)], axis=1)
    m = (e_src + e_dst
         + jnp.dot(ef.astype(jnp.bfloat16), weg_ref[...].astype(jnp.bfloat16),
                   preferred_element_type=jnp.float32)
         + beg_ref[...])
    ln = _layernorm(m, ge_ref[...], bee_ref[...])
    y = ef + ln * jax.nn.sigmoid(ln)
    sig = jax.nn.sigmoid(y)
    y_ref[...] = y
    sig_ref[...] = sig
    t_ref[...] = bh * sig


def _edge_stage(edge_feats, g_src_bh, g_dst, W_eg, b_eg, g_e, be_e,
                block=4000):
    e, d = edge_feats.shape
    row = lambda i: (i, 0)
    fixed = lambda i: (0, 0)
    out_sd = jax.ShapeDtypeStruct((e, d), jnp.float32)
    return pl.pallas_call(
        _edge_body,
        grid=(e // block,),
        in_specs=[
            pl.BlockSpec((block, d), row),
            pl.BlockSpec((block, d), row),
            pl.BlockSpec((block, d // 2), row),
            pl.BlockSpec((d, d), fixed),
            pl.BlockSpec((1, d), fixed),
            pl.BlockSpec((1, d), fixed),
            pl.BlockSpec((1, d), fixed),
        ],
        out_specs=[
            pl.BlockSpec((block, d), row),
            pl.BlockSpec((block, d), row),
            pl.BlockSpec((block, d), row),
        ],
        out_shape=[out_sd, out_sd, out_sd],
    )(edge_feats, g_src_bh, g_dst, W_eg, b_eg, g_e, be_e)


def _node_out_body(nf_ref, xs_ref, ssh_ref, ss_ref, gn_ref, ben_ref, x_ref):
    h = ssh_ref[...] / (ss_ref[...] + 1e-6)
    v = xs_ref[...] + h
    ln = _layernorm(v, gn_ref[...], ben_ref[...])
    x_ref[...] = nf_ref[...] + ln * jax.nn.sigmoid(ln)


def _node_out_stage(node_feats, xs, ssh, ss, g_n, be_n, block=2000):
    n, d = node_feats.shape
    row = lambda i: (i, 0)
    fixed = lambda i: (0, 0)
    return pl.pallas_call(
        _node_out_body,
        grid=(n // block,),
        in_specs=[
            pl.BlockSpec((block, d), row),
            pl.BlockSpec((block, d), row),
            pl.BlockSpec((block, d), row),
            pl.BlockSpec((block, d), row),
            pl.BlockSpec((1, d), fixed),
            pl.BlockSpec((1, d), fixed),
        ],
        out_specs=pl.BlockSpec((block, d), row),
        out_shape=jax.ShapeDtypeStruct((n, d), jnp.float32),
    )(node_feats, xs, ssh, ss, g_n, be_n)


def _pad_rows(x, rows):
    if x.shape[0] >= rows:
        return x
    return jnp.concatenate(
        [x, jnp.zeros((rows - x.shape[0],) + x.shape[1:], x.dtype)], axis=0)


def _gather_split(idx, chunk):
    e = idx.shape[0]
    n_chunks = e // chunk
    per, rem = n_chunks // _NW, n_chunks % _NW
    maxc = per + (1 if rem else 0)
    if rem:
        idx = jnp.concatenate([idx, jnp.zeros((chunk,), idx.dtype)])
    return n_chunks, per, rem, maxc, idx


def _gather_phase(table_hbm, idx_hbm, out_hbm, idx_v, rows0, rows1,
                  gsem, osem0, osem1, chunk, per, rem, maxc):
    wid = lax.axis_index("s") * _SC_CORES + lax.axis_index("c")
    start = wid * per + jnp.minimum(wid, rem)
    count = per + jnp.where(wid < rem, 1, 0)
    pltpu.sync_copy(idx_hbm.at[pl.ds(start * chunk, maxc * chunk)], idx_v)
    npairs = (count + 1) // 2

    def out_slice(k_):
        return out_hbm.at[pl.ds((start + k_) * chunk, chunk)]

    @pl.loop(0, npairs)
    def _(p):
        k0 = 2 * p
        k1 = k0 + 1

        @pl.when(p > 0)
        def _():
            pltpu.make_async_copy(rows0, out_slice(0), osem0).wait()

        pltpu.async_copy(
            table_hbm.at[idx_v.at[pl.ds(k0 * chunk, chunk)]], rows0, gsem
        ).wait()
        pltpu.async_copy(rows0, out_slice(k0), osem0)

        @pl.when(k1 < count)
        def _():
            @pl.when(p > 0)
            def _():
                pltpu.make_async_copy(rows1, out_slice(0), osem1).wait()

            pltpu.async_copy(
                table_hbm.at[idx_v.at[pl.ds(k1 * chunk, chunk)]], rows1, gsem
            ).wait()
            pltpu.async_copy(rows1, out_slice(k1), osem1)

    @pl.when(count > 0)
    def _():
        pltpu.make_async_copy(rows0, out_slice(0), osem0).wait()

    @pl.when(count > 1)
    def _():
        pltpu.make_async_copy(rows1, out_slice(0), osem1).wait()


def _sc_gather(table, idx, chunk):
    e = idx.shape[0]
    n, d = table.shape
    _, per, rem, maxc, idx = _gather_split(idx, chunk)
    mesh = plsc.VectorSubcoreMesh(core_axis_name="c", subcore_axis_name="s")

    @functools.partial(
        pl.kernel,
        mesh=mesh,
        out_type=jax.ShapeDtypeStruct((e, d), table.dtype),
        scratch_types=[
            pltpu.VMEM((maxc * chunk,), jnp.int32),
            pltpu.VMEM((chunk, d), table.dtype),
            pltpu.VMEM((chunk, d), table.dtype),
            pltpu.SemaphoreType.DMA,
            pltpu.SemaphoreType.DMA,
            pltpu.SemaphoreType.DMA,
        ],
    )
    def k(t_hbm, i_hbm, o_hbm, i_v, r0, r1, gsem, osem0, osem1):
        _gather_phase(t_hbm, i_hbm, o_hbm, i_v, r0, r1,
                      gsem, osem0, osem1, chunk, per, rem, maxc)

    return k(table, idx)


def _sc_segment_sum(vals, idx, n):
    e, d = vals.shape
    dh = d // _SC_CORES
    vals3 = vals.reshape(e, _SC_CORES, dh)
    n_chunks = e // _CHUNK
    rows_per_sub = n // _SC_SUBCORES
    mesh = plsc.VectorSubcoreMesh(core_axis_name="c", subcore_axis_name="s")

    stride = max(-(-n_chunks // (_SC_SUBCORES * 8)) * 8, 8)
    idx2 = _pad_rows(idx.reshape(n_chunks, _CHUNK), _SC_SUBCORES * stride)
    zeros = jnp.zeros((rows_per_sub, dh), jnp.float32)

    @functools.partial(
        pl.kernel,
        mesh=mesh,
        out_type=jax.ShapeDtypeStruct((n, _SC_CORES, dh), jnp.float32),
        scratch_types=[
            pltpu.VMEM((stride, _CHUNK), jnp.int32),
            pltpu.VMEM((_CHUNK, dh), jnp.float32),
            pltpu.VMEM((_CHUNK, dh), jnp.float32),
            pltpu.VMEM_SHARED((n, dh), jnp.float32),
            pltpu.SemaphoreType.DMA,
            pltpu.SemaphoreType.DMA,
        ],
    )
    def k(vals_hbm, idx_hbm, z_hbm, out_hbm, idx_v, vals0, vals1, acc_sh,
          vsem0, vsem1):
        c = lax.axis_index("c")
        s = lax.axis_index("s")
        start = s * stride
        count = jnp.clip(n_chunks - start, 0, stride)
        my_rows = pl.ds(s * rows_per_sub, rows_per_sub)
        pltpu.sync_copy(idx_hbm.at[pl.ds(start, stride)], idx_v)
        pltpu.sync_copy(z_hbm, acc_sh.at[my_rows])
        plsc.subcore_barrier()

        def vals_slice(k_):
            return vals_hbm.at[pl.ds((start + k_) * _CHUNK, _CHUNK), c]

        @pl.when(count > 0)
        def _():
            pltpu.async_copy(vals_slice(0), vals0, vsem0)

        npairs = (count + 1) // 2

        @pl.loop(0, npairs)
        def _(p):
            k0 = 2 * p
            k1 = k0 + 1
            k2 = k0 + 2

            @pl.when(k1 < count)
            def _():
                pltpu.async_copy(vals_slice(k1), vals1, vsem1)

            pltpu.make_async_copy(vals_slice(0), vals0, vsem0).wait()
            pltpu.sync_copy(vals0, acc_sh.at[idx_v.at[k0]], add=True)

            @pl.when(k2 < count)
            def _():
                pltpu.async_copy(vals_slice(k2), vals0, vsem0)

            @pl.when(k1 < count)
            def _():
                pltpu.make_async_copy(vals_slice(0), vals1, vsem1).wait()
                pltpu.sync_copy(vals1, acc_sh.at[idx_v.at[k1]], add=True)

        plsc.subcore_barrier()
        pltpu.sync_copy(acc_sh.at[my_rows], out_hbm.at[my_rows, c])

    return k(vals3, idx2, zeros).reshape(n, d)


def kernel(node_feats, edge_feats, edge_index, W_sg, b_sg, W_dg, b_dg,
           W_eg, b_eg, W_su, b_su, W_du, b_du, g_e, be_e, g_n, be_n):
    n, d = node_feats.shape
    src = edge_index[0].astype(jnp.int32)
    dst = edge_index[1].astype(jnp.int32)

    W4 = jnp.concatenate([W_sg, W_du, W_dg, W_su], axis=1)
    b4 = jnp.concatenate([b_sg, b_du, b_dg, b_su]).reshape(1, 4 * d)
    src_tab, e_dst, xs = _node_proj4(node_feats, W4, b4)

    g_src_bh = _sc_gather(src_tab, src, 128)
    g_dst = _sc_gather(e_dst, dst, 128)

    y, sigma, t = _edge_stage(
        edge_feats, g_src_bh, g_dst, W_eg,
        b_eg.reshape(1, d), g_e.reshape(1, d), be_e.reshape(1, d))

    ssh = _sc_segment_sum(t, dst, n)
    ss = _sc_segment_sum(sigma, dst, n)

    x = _node_out_stage(node_feats, xs, ssh, ss,
                        g_n.reshape(1, d), be_n.reshape(1, d))
    return (x, y)
